# baseline (speedup 1.0000x reference)
import functools

import jax
import jax.numpy as jnp
import numpy as np
from jax import lax
from jax.experimental import pallas as pl
from jax.experimental.pallas import tpu as pltpu
from jax.experimental.pallas import tpu_sc as plsc

NUM_CORES = 2
NUM_SUBCORES = 16
NW = NUM_CORES * NUM_SUBCORES
LANES = 16

F = 26
V = 100000
D = 64
B = 4096
R = B * F
ROWS_W = R // NW
CHUNK = 128
NCHUNK = ROWS_W // CHUNK

_OFFS = np.asarray(
    (np.arange(ROWS_W, dtype=np.int64) % F) * V, dtype=np.int32
).reshape(NCHUNK, CHUNK)

_mesh = plsc.VectorSubcoreMesh(core_axis_name="c", subcore_axis_name="s")


@functools.partial(
    pl.kernel,
    mesh=_mesh,
    compiler_params=pltpu.CompilerParams(use_tc_tiling_on_sc=False),
    out_type=jax.ShapeDtypeStruct((R, D), jnp.float32),
    scratch_types=[
        pltpu.VMEM((NCHUNK, CHUNK), jnp.int32),
        pltpu.VMEM((NCHUNK, CHUNK), jnp.int32),
        pltpu.VMEM((CHUNK, D), jnp.float32),
        pltpu.VMEM((CHUNK, D), jnp.float32),
        pltpu.SemaphoreType.DMA,
        pltpu.SemaphoreType.DMA,
    ],
)
def _sc_gather(idx_hbm, offs_hbm, tab_hbm, out_hbm,
               idx_v, offs_v, buf0, buf1, gsem0, gsem1):
    wid = lax.axis_index("s") * NUM_CORES + lax.axis_index("c")
    out_base = wid * ROWS_W

    pltpu.sync_copy(idx_hbm.at[wid], idx_v)
    pltpu.sync_copy(offs_hbm, offs_v)

    def add_offsets(c):
        for j in range(CHUNK // LANES):
            sl = pl.ds(j * LANES, LANES)
            idx_v[c, sl] = idx_v[c, sl] + offs_v[c, sl]

    def gather_start(c, buf, sem):
        pltpu.make_async_copy(tab_hbm.at[idx_v.at[c]], buf, sem).start()

    def gather_wait(c, buf, sem):
        pltpu.make_async_copy(tab_hbm.at[idx_v.at[c]], buf, sem).wait()

    def scatter(c, buf):
        pltpu.sync_copy(buf, out_hbm.at[pl.ds(out_base + c * CHUNK, CHUNK)])

    add_offsets(0)
    add_offsets(1)
    gather_start(0, buf0, gsem0)
    gather_start(1, buf1, gsem1)

    def loop_body(i, _):
        for b, (buf, sem) in enumerate(((buf0, gsem0), (buf1, gsem1))):
            c = 2 * i + b
            gather_wait(c, buf, sem)
            scatter(c, buf)
            add_offsets(c + 2)
            gather_start(c + 2, buf, sem)
        return 0

    lax.fori_loop(0, (NCHUNK - 2) // 2, loop_body, 0)

    for b, (buf, sem) in enumerate(((buf0, gsem0), (buf1, gsem1))):
        c = NCHUNK - 2 + b
        gather_wait(c, buf, sem)
        scatter(c, buf)


def kernel(X, tables):
    idx = jnp.asarray(X, jnp.int32).reshape(NW, NCHUNK, CHUNK)
    tab = tables.reshape(F * V, D)
    offs = jnp.asarray(_OFFS)
    out_flat = _sc_gather(idx, offs, tab)
    return out_flat.reshape(B, F, D)


# speedup vs baseline: 1.0046x; 1.0046x over previous
import functools

import jax
import jax.numpy as jnp
import numpy as np
from jax import lax
from jax.experimental import pallas as pl
from jax.experimental.pallas import tpu as pltpu
from jax.experimental.pallas import tpu_sc as plsc

NUM_CORES = 2
NUM_SUBCORES = 16
NW = NUM_CORES * NUM_SUBCORES
LANES = 16

F = 26
V = 100000
D = 64
B = 4096
R = B * F
ROWS_W = R // NW
CHUNK = 832
NCHUNK = ROWS_W // CHUNK

_OFFS = np.asarray(
    (np.arange(ROWS_W, dtype=np.int64) % F) * V, dtype=np.int32
).reshape(NCHUNK, CHUNK)

_mesh = plsc.VectorSubcoreMesh(core_axis_name="c", subcore_axis_name="s")


@functools.partial(
    pl.kernel,
    mesh=_mesh,
    compiler_params=pltpu.CompilerParams(use_tc_tiling_on_sc=False),
    out_type=jax.ShapeDtypeStruct((R, D), jnp.float32),
    scratch_types=[
        pltpu.VMEM((NCHUNK, CHUNK), jnp.int32),
        pltpu.VMEM((NCHUNK, CHUNK), jnp.int32),
        pltpu.VMEM((CHUNK, D), jnp.float32),
        pltpu.VMEM((CHUNK, D), jnp.float32),
        pltpu.SemaphoreType.DMA,
        pltpu.SemaphoreType.DMA,
    ],
)
def _sc_gather(idx_hbm, offs_hbm, tab_hbm, out_hbm,
               idx_v, offs_v, buf0, buf1, gsem0, gsem1):
    wid = lax.axis_index("s") * NUM_CORES + lax.axis_index("c")
    out_base = wid * ROWS_W

    pltpu.sync_copy(idx_hbm.at[wid], idx_v)
    pltpu.sync_copy(offs_hbm, offs_v)

    def add_offsets(c):
        for j in range(CHUNK // LANES):
            sl = pl.ds(j * LANES, LANES)
            idx_v[c, sl] = idx_v[c, sl] + offs_v[c, sl]

    def gather_start(c, buf, sem):
        pltpu.make_async_copy(tab_hbm.at[idx_v.at[c]], buf, sem).start()

    def gather_wait(c, buf, sem):
        pltpu.make_async_copy(tab_hbm.at[idx_v.at[c]], buf, sem).wait()

    def scatter(c, buf):
        pltpu.sync_copy(buf, out_hbm.at[pl.ds(out_base + c * CHUNK, CHUNK)])

    add_offsets(0)
    add_offsets(1)
    gather_start(0, buf0, gsem0)
    gather_start(1, buf1, gsem1)

    def loop_body(i, _):
        for b, (buf, sem) in enumerate(((buf0, gsem0), (buf1, gsem1))):
            c = 2 * i + b
            gather_wait(c, buf, sem)
            scatter(c, buf)
            add_offsets(c + 2)
            gather_start(c + 2, buf, sem)
        return 0

    lax.fori_loop(0, (NCHUNK - 2) // 2, loop_body, 0)

    for b, (buf, sem) in enumerate(((buf0, gsem0), (buf1, gsem1))):
        c = NCHUNK - 2 + b
        gather_wait(c, buf, sem)
        scatter(c, buf)


def kernel(X, tables):
    idx = jnp.asarray(X, jnp.int32).reshape(NW, NCHUNK, CHUNK)
    tab = tables.reshape(F * V, D)
    offs = jnp.asarray(_OFFS)
    out_flat = _sc_gather(idx, offs, tab)
    return out_flat.reshape(B, F, D)
